# Initial kernel scaffold; baseline (speedup 1.0000x reference)
import jax
import jax.numpy as jnp
from jax.experimental import pallas as pl


def kernel(vis_feat, text_feat, vis_memory, text_memory, idx, slct_idx):
    raise NotImplementedError("write your pallas kernel here")


# speedup vs baseline: 12.7564x; 12.7564x over previous
import functools

import jax
import jax.numpy as jnp
from jax import lax
from jax.experimental import pallas as pl
from jax.experimental.pallas import tpu as pltpu
from jax.experimental.pallas import tpu_sc as plsc

_N = 100000
_EMB = 128
_K1 = 4097
_T = 0.07
_M = 0.5
_BS = 128
_LANES = 16

_ROWS_BLK = 2048
_N_BLK = (_N + _ROWS_BLK - 1) // _ROWS_BLK
_KP = 4112
_N_TILES = 32
_ROWS_PER_TILE = _BS // _N_TILES


def _tc_scores_body(vf_ref, tf_ref, vmem_ref, tmem_ref,
                    vs_ref, ts_ref, vcopy_ref, tcopy_ref):
    vf = vf_ref[:, :]
    tf = tf_ref[:, :]
    vn = vf / jnp.maximum(
        jnp.sqrt(jnp.sum(vf * vf, axis=1, keepdims=True)), 1e-12)
    tn = tf / jnp.maximum(
        jnp.sqrt(jnp.sum(tf * tf, axis=1, keepdims=True)), 1e-12)
    vblk = vmem_ref[:, :]
    tblk = tmem_ref[:, :]
    dn = (((1,), (1,)), ((), ()))
    vs_ref[:, :] = lax.dot_general(vn, tblk, dn,
                                   preferred_element_type=jnp.float32)
    ts_ref[:, :] = lax.dot_general(tn, vblk, dn,
                                   preferred_element_type=jnp.float32)
    vcopy_ref[:, :] = vblk
    tcopy_ref[:, :] = tblk


_scores_call = pl.pallas_call(
    _tc_scores_body,
    grid=(_N_BLK,),
    in_specs=[
        pl.BlockSpec((_BS, _EMB), lambda g: (0, 0)),
        pl.BlockSpec((_BS, _EMB), lambda g: (0, 0)),
        pl.BlockSpec((_ROWS_BLK, _EMB), lambda g: (g, 0)),
        pl.BlockSpec((_ROWS_BLK, _EMB), lambda g: (g, 0)),
    ],
    out_specs=[
        pl.BlockSpec((_BS, _ROWS_BLK), lambda g: (0, g)),
        pl.BlockSpec((_BS, _ROWS_BLK), lambda g: (0, g)),
        pl.BlockSpec((_ROWS_BLK, _EMB), lambda g: (g, 0)),
        pl.BlockSpec((_ROWS_BLK, _EMB), lambda g: (g, 0)),
    ],
    out_shape=[
        jax.ShapeDtypeStruct((_BS, _N), jnp.float32),
        jax.ShapeDtypeStruct((_BS, _N), jnp.float32),
        jax.ShapeDtypeStruct((_N, _EMB), jnp.float32),
        jax.ShapeDtypeStruct((_N, _EMB), jnp.float32),
    ],
)


@functools.partial(
    pl.kernel,
    out_type=[
        jax.ShapeDtypeStruct((_BS, _KP), jnp.float32),
        jax.ShapeDtypeStruct((_BS, _KP), jnp.float32),
    ],
    mesh=plsc.VectorSubcoreMesh(core_axis_name="c", subcore_axis_name="s"),
    compiler_params=pltpu.CompilerParams(needs_layout_passes=False),
    scratch_types=[
        pltpu.VMEM((_N,), jnp.float32),
        pltpu.VMEM((_KP,), jnp.int32),
        pltpu.VMEM((_KP,), jnp.float32),
    ],
)
def _sc_gather(vs_hbm, ts_hbm, slct_hbm, gv_hbm, gt_hbm,
               scores_v, idx_v, out_v):
    wid = lax.axis_index("s") * 2 + lax.axis_index("c")
    for r in range(_ROWS_PER_TILE):
        b = wid * _ROWS_PER_TILE + r
        pltpu.sync_copy(slct_hbm.at[b], idx_v)
        for src_hbm, dst_hbm in ((vs_hbm, gv_hbm), (ts_hbm, gt_hbm)):
            pltpu.sync_copy(src_hbm.at[b], scores_v)

            def gbody(i, _):
                off = pl.multiple_of(i * _LANES, _LANES)
                ids = idx_v[pl.ds(off, _LANES)]
                out_v[pl.ds(off, _LANES)] = plsc.load_gather(scores_v, [ids])
                return 0

            lax.fori_loop(0, _KP // _LANES, gbody, 0)
            pltpu.sync_copy(out_v, dst_hbm.at[b])


def _tc_norm_body(gv_ref, gt_ref, vo_ref, to_ref):
    col = lax.broadcasted_iota(jnp.int32, (_BS, _KP), 1)
    valid = col < _K1
    ev = jnp.where(valid, jnp.exp(gv_ref[:, :] * (1.0 / _T)), 0.0)
    et = jnp.where(valid, jnp.exp(gt_ref[:, :] * (1.0 / _T)), 0.0)
    cnt = float(_BS * _K1)
    zv = jnp.sum(ev) * (float(_N) / cnt)
    zt = jnp.sum(et) * (float(_N) / cnt)
    vo_ref[:, :] = (ev * (1.0 / zv))[:, :_K1]
    to_ref[:, :] = (et * (1.0 / zt))[:, :_K1]


_norm_call = pl.pallas_call(
    _tc_norm_body,
    in_specs=[
        pl.BlockSpec((_BS, _KP), lambda: (0, 0)),
        pl.BlockSpec((_BS, _KP), lambda: (0, 0)),
    ],
    out_specs=[
        pl.BlockSpec((_BS, _K1), lambda: (0, 0)),
        pl.BlockSpec((_BS, _K1), lambda: (0, 0)),
    ],
    out_shape=[
        jax.ShapeDtypeStruct((_BS, _K1), jnp.float32),
        jax.ShapeDtypeStruct((_BS, _K1), jnp.float32),
    ],
)


def _tc_scatter_body(idx_ref, vf_ref, tf_ref, vrow_ref, trow_ref,
                     vany, tany, vout_ref, tout_ref):
    del idx_ref, vany, tany
    vfb = vf_ref[0, :, :]
    tfb = tf_ref[0, :, :]
    vnb = vfb / jnp.maximum(jnp.sqrt(jnp.sum(vfb * vfb)), 1e-12)
    tnb = tfb / jnp.maximum(jnp.sqrt(jnp.sum(tfb * tfb)), 1e-12)
    vnew = vrow_ref[0, :, :] * _M + vnb * (1.0 - _M)
    tnew = trow_ref[0, :, :] * _M + tnb * (1.0 - _M)
    vout_ref[0, :, :] = vnew / jnp.maximum(
        jnp.sqrt(jnp.sum(vnew * vnew)), 1e-12)
    tout_ref[0, :, :] = tnew / jnp.maximum(
        jnp.sqrt(jnp.sum(tnew * tnew)), 1e-12)


_scatter_call = pl.pallas_call(
    _tc_scatter_body,
    grid_spec=pltpu.PrefetchScalarGridSpec(
        num_scalar_prefetch=1,
        grid=(_BS,),
        in_specs=[
            pl.BlockSpec((1, 1, _EMB), lambda b, idx_ref: (b, 0, 0)),
            pl.BlockSpec((1, 1, _EMB), lambda b, idx_ref: (b, 0, 0)),
            pl.BlockSpec((1, 1, _EMB), lambda b, idx_ref: (idx_ref[b], 0, 0)),
            pl.BlockSpec((1, 1, _EMB), lambda b, idx_ref: (idx_ref[b], 0, 0)),
            pl.BlockSpec(memory_space=pltpu.MemorySpace.HBM),
            pl.BlockSpec(memory_space=pltpu.MemorySpace.HBM),
        ],
        out_specs=[
            pl.BlockSpec((1, 1, _EMB), lambda b, idx_ref: (idx_ref[b], 0, 0)),
            pl.BlockSpec((1, 1, _EMB), lambda b, idx_ref: (idx_ref[b], 0, 0)),
        ],
    ),
    out_shape=[
        jax.ShapeDtypeStruct((_N, 1, _EMB), jnp.float32),
        jax.ShapeDtypeStruct((_N, 1, _EMB), jnp.float32),
    ],
    input_output_aliases={5: 0, 6: 1},
)


def kernel(vis_feat, text_feat, vis_memory, text_memory, idx, slct_idx):
    idx = idx.astype(jnp.int32)
    slct = slct_idx.astype(jnp.int32).at[:, 0].set(idx)
    slct_p = jnp.pad(slct, ((0, 0), (0, _KP - _K1)))
    vs, ts, vcopy, tcopy = _scores_call(
        vis_feat, text_feat, vis_memory, text_memory)
    gv, gt = _sc_gather(vs, ts, slct_p)
    vis_out, text_out = _norm_call(gv, gt)
    vmem_new, tmem_new = _scatter_call(
        idx,
        vis_feat.reshape(_BS, 1, _EMB),
        text_feat.reshape(_BS, 1, _EMB),
        vis_memory.reshape(_N, 1, _EMB),
        text_memory.reshape(_N, 1, _EMB),
        vcopy.reshape(_N, 1, _EMB),
        tcopy.reshape(_N, 1, _EMB),
    )
    return (vis_out, text_out,
            vmem_new.reshape(_N, _EMB), tmem_new.reshape(_N, _EMB))
